# baseline (speedup 1.0000x reference)
import functools

import jax
import jax.numpy as jnp
from jax.experimental import pallas as pl

DIM = 1024
HEADS = 16
HD = DIM // HEADS
NEXP = 8
DSTEP = DIM // NEXP
SCALE = HD ** -0.5


def _qkv_kernel(x_ref, em_ref, w_ref, o_ref):
    x = x_ref[...]
    em = em_ref[0, 0]
    d_tok = (em + 1) * DSTEP
    col = jax.lax.broadcasted_iota(jnp.int32, x.shape, 1)
    xm = jnp.where(col < d_tok[:, None], x, 0.0).astype(jnp.bfloat16)
    acc = jax.lax.dot_general(w_ref[...], xm, (((1,), (1,)), ((), ())),
                              preferred_element_type=jnp.float32)
    o_ref[...] = acc.astype(jnp.bfloat16)


def _attn_kernel(q_ref, k_ref, v_ref, o_ref, *, bq, chunk):
    k = k_ref[...]
    v = v_ref[...]
    n = v.shape[1]
    v_ext = jnp.concatenate(
        [v, jnp.ones((8, n), jnp.bfloat16)], axis=0)
    kf = k.astype(jnp.float32)
    mk2 = jnp.max(jnp.sum(kf * kf, axis=0))
    for c in range(bq // chunk):
        q = q_ref[:, c * chunk:(c + 1) * chunk]
        qf = q.astype(jnp.float32)
        q2 = jnp.sum(qf * qf, axis=0, keepdims=True)
        bound = jnp.sqrt(q2 * mk2).reshape(chunk, 1)
        s = jax.lax.dot_general(q, k, (((0,), (0,)), ((), ())),
                                preferred_element_type=jnp.float32)
        p = jnp.exp(s - bound).astype(jnp.bfloat16)
        o = jax.lax.dot_general(v_ext, p, (((1,), (1,)), ((), ())),
                                preferred_element_type=jnp.float32)
        l = o[HD:HD + 1, :]
        o = o[:HD, :] / l
        o_ref[:, c * chunk:(c + 1) * chunk] = o.astype(jnp.bfloat16)


def _proj_kernel(x_ref, em_ref, w_ref, b_ref, o_ref):
    x = x_ref[...]
    y = jax.lax.dot_general(x, w_ref[...], (((0,), (1,)), ((), ())),
                            preferred_element_type=jnp.float32)
    y = y + b_ref[...][None, :]
    em = em_ref[0, 0]
    d_tok = (em + 1) * DSTEP
    col = jax.lax.broadcasted_iota(jnp.int32, y.shape, 1)
    o_ref[...] = jnp.where(col < d_tok[:, None], y, 0.0)


def kernel(input_tokens, expert_mask, Wqkv, Wproj, bproj):
    B, N, D = input_tokens.shape
    R = 512
    BQ = 2048
    CHUNK = 128
    nrow = (B * N) // R
    nq = N // BQ

    x2 = input_tokens.reshape(B * N, D)
    em_r = expert_mask.reshape(nrow, 1, R)
    wqkv_b = jnp.concatenate([Wqkv[:D] * SCALE, Wqkv[D:]],
                             axis=0).astype(jnp.bfloat16)
    wproj_b = Wproj.astype(jnp.bfloat16)

    qkv = pl.pallas_call(
        _qkv_kernel,
        grid=(nrow,),
        in_specs=[
            pl.BlockSpec((R, D), lambda i: (i, 0)),
            pl.BlockSpec((1, 1, R), lambda i: (i, 0, 0)),
            pl.BlockSpec((3 * D, D), lambda i: (0, 0)),
        ],
        out_specs=pl.BlockSpec((3 * D, R), lambda i: (0, i)),
        out_shape=jax.ShapeDtypeStruct((3 * D, B * N), jnp.bfloat16),
    )(x2, em_r, wqkv_b)

    attn_body = functools.partial(_attn_kernel, bq=BQ, chunk=CHUNK)
    x_fm = pl.pallas_call(
        attn_body,
        grid=(B, HEADS, nq),
        in_specs=[
            pl.BlockSpec((HD, BQ), lambda b, h, qi: (h, b * nq + qi)),
            pl.BlockSpec((HD, N), lambda b, h, qi: (HEADS + h, b)),
            pl.BlockSpec((HD, N), lambda b, h, qi: (2 * HEADS + h, b)),
        ],
        out_specs=pl.BlockSpec((HD, BQ), lambda b, h, qi: (h, b * nq + qi)),
        out_shape=jax.ShapeDtypeStruct((D, B * N), jnp.bfloat16),
    )(qkv, qkv, qkv)

    y = pl.pallas_call(
        _proj_kernel,
        grid=(nrow,),
        in_specs=[
            pl.BlockSpec((D, R), lambda i: (0, i)),
            pl.BlockSpec((1, 1, R), lambda i: (i, 0, 0)),
            pl.BlockSpec((D, D), lambda i: (0, 0)),
            pl.BlockSpec((D,), lambda i: (0,)),
        ],
        out_specs=pl.BlockSpec((R, D), lambda i: (i, 0)),
        out_shape=jax.ShapeDtypeStruct((B * N, D), jnp.float32),
    )(x_fm, em_r, wproj_b, bproj)

    return y.reshape(B, N, D)


# speedup vs baseline: 1.0633x; 1.0633x over previous
import functools

import jax
import jax.numpy as jnp
from jax.experimental import pallas as pl

DIM = 1024
HEADS = 16
HD = DIM // HEADS
NEXP = 8
DSTEP = DIM // NEXP
SCALE = HD ** -0.5


def _qkv_kernel(x_ref, em_ref, w_ref, o_ref):
    x = x_ref[...]
    em = em_ref[0, 0]
    d_tok = (em + 1) * DSTEP
    col = jax.lax.broadcasted_iota(jnp.int32, x.shape, 1)
    xm = jnp.where(col < d_tok[:, None], x, 0.0).astype(jnp.bfloat16)
    acc = jax.lax.dot_general(w_ref[...], xm, (((1,), (1,)), ((), ())),
                              preferred_element_type=jnp.float32)
    o_ref[...] = acc.astype(jnp.bfloat16)


def _attn_kernel(q_ref, k_ref, v_ref, o_ref, *, bq, chunk):
    k = k_ref[...]
    v = v_ref[...]
    n = v.shape[1]
    v_ext = jnp.concatenate(
        [v, jnp.ones((8, n), jnp.bfloat16)], axis=0)
    kf = k.astype(jnp.float32)
    mk2 = jnp.max(jnp.sum(kf * kf, axis=0))
    for c in range(bq // chunk):
        q = q_ref[:, c * chunk:(c + 1) * chunk]
        qf = q.astype(jnp.float32)
        q2 = jnp.sum(qf * qf, axis=0, keepdims=True)
        bound = jnp.sqrt(q2 * mk2).reshape(chunk, 1)
        s = jax.lax.dot_general(q, k, (((0,), (0,)), ((), ())),
                                preferred_element_type=jnp.float32)
        p = jnp.exp(s - bound).astype(jnp.bfloat16)
        o = jax.lax.dot_general(v_ext, p, (((1,), (1,)), ((), ())),
                                preferred_element_type=jnp.float32)
        l = o[HD:HD + 1, :]
        o = o[:HD, :] / l
        o_ref[:, c * chunk:(c + 1) * chunk] = o.astype(jnp.bfloat16)


def _proj_kernel(x_ref, em_ref, w_ref, b_ref, o_ref):
    x = x_ref[...]
    y = jax.lax.dot_general(x, w_ref[...], (((0,), (1,)), ((), ())),
                            preferred_element_type=jnp.float32)
    y = y + b_ref[...][None, :]
    em = em_ref[0, 0]
    d_tok = (em + 1) * DSTEP
    col = jax.lax.broadcasted_iota(jnp.int32, y.shape, 1)
    o_ref[...] = jnp.where(col < d_tok[:, None], y, 0.0)


def kernel(input_tokens, expert_mask, Wqkv, Wproj, bproj):
    B, N, D = input_tokens.shape
    R = 1024
    BQ = 2048
    CHUNK = 256
    nrow = (B * N) // R
    nq = N // BQ

    x2 = input_tokens.reshape(B * N, D)
    em_r = expert_mask.reshape(nrow, 1, R)
    wqkv_b = jnp.concatenate([Wqkv[:D] * SCALE, Wqkv[D:]],
                             axis=0).astype(jnp.bfloat16)
    wproj_b = Wproj.astype(jnp.bfloat16)

    qkv = pl.pallas_call(
        _qkv_kernel,
        grid=(nrow,),
        in_specs=[
            pl.BlockSpec((R, D), lambda i: (i, 0)),
            pl.BlockSpec((1, 1, R), lambda i: (i, 0, 0)),
            pl.BlockSpec((3 * D, D), lambda i: (0, 0)),
        ],
        out_specs=pl.BlockSpec((3 * D, R), lambda i: (0, i)),
        out_shape=jax.ShapeDtypeStruct((3 * D, B * N), jnp.bfloat16),
    )(x2, em_r, wqkv_b)

    attn_body = functools.partial(_attn_kernel, bq=BQ, chunk=CHUNK)
    x_fm = pl.pallas_call(
        attn_body,
        grid=(B, HEADS, nq),
        in_specs=[
            pl.BlockSpec((HD, BQ), lambda b, h, qi: (h, b * nq + qi)),
            pl.BlockSpec((HD, N), lambda b, h, qi: (HEADS + h, b)),
            pl.BlockSpec((HD, N), lambda b, h, qi: (2 * HEADS + h, b)),
        ],
        out_specs=pl.BlockSpec((HD, BQ), lambda b, h, qi: (h, b * nq + qi)),
        out_shape=jax.ShapeDtypeStruct((D, B * N), jnp.bfloat16),
    )(qkv, qkv, qkv)

    y = pl.pallas_call(
        _proj_kernel,
        grid=(nrow,),
        in_specs=[
            pl.BlockSpec((D, R), lambda i: (0, i)),
            pl.BlockSpec((1, 1, R), lambda i: (i, 0, 0)),
            pl.BlockSpec((D, D), lambda i: (0, 0)),
            pl.BlockSpec((D,), lambda i: (0,)),
        ],
        out_specs=pl.BlockSpec((R, D), lambda i: (i, 0)),
        out_shape=jax.ShapeDtypeStruct((B * N, D), jnp.float32),
    )(x_fm, em_r, wproj_b, bproj)

    return y.reshape(B, N, D)
